# baseline (speedup 1.0000x reference)
import jax
import jax.numpy as jnp
from jax.experimental import pallas as pl
from jax.experimental.pallas import tpu as pltpu

R = 128
C = 32768
BC_R = 4096
BC_W = 8192
NB_R = C // BC_R
NB_W = C // BC_W


def _body(x_ref, out_ref, run_max, run_idx, col_scratch):
    t = pl.program_id(0)

    @pl.when(t == 0)
    def _():
        col_scratch[...] = jax.lax.broadcasted_iota(jnp.int32, (R, BC_W), 1)

    @pl.when(t < NB_R)
    def _():
        x = x_ref[...]
        m = jnp.max(x, axis=1, keepdims=True)
        loc = jnp.argmax(x, axis=1).astype(jnp.int32).reshape(R, 1) + t * BC_R

        @pl.when(t == 0)
        def _():
            run_max[...] = m
            run_idx[...] = loc

        @pl.when(t > 0)
        def _():
            better = m > run_max[...]
            run_idx[...] = jnp.where(better, loc, run_idx[...])
            run_max[...] = jnp.maximum(m, run_max[...])

    @pl.when(t >= NB_R)
    def _():
        j = t - NB_R
        idx_s = run_idx[...] - j * BC_W
        out_ref[...] = (col_scratch[...] == idx_s).astype(jnp.float32)


@jax.jit
def kernel(Xsoft, P):
    del P
    return pl.pallas_call(
        _body,
        grid=(NB_R + NB_W,),
        in_specs=[
            pl.BlockSpec((R, BC_R), lambda t: (0, jnp.minimum(t, NB_R - 1))),
        ],
        out_specs=pl.BlockSpec(
            (R, BC_W), lambda t: (0, jnp.maximum(t - NB_R, 0))
        ),
        out_shape=jax.ShapeDtypeStruct((R, C), jnp.float32),
        scratch_shapes=[
            pltpu.VMEM((R, 1), jnp.float32),
            pltpu.VMEM((R, 1), jnp.int32),
            pltpu.VMEM((R, BC_W), jnp.int32),
        ],
    )(Xsoft)


# speedup vs baseline: 1.1159x; 1.1159x over previous
import jax
import jax.numpy as jnp
from jax.experimental import pallas as pl
from jax.experimental.pallas import tpu as pltpu

R = 128
C = 32768
BC_R = 16384
BC_W = 8192
NB_R = C // BC_R
NB_W = C // BC_W


def _body(x_ref, out_ref, run_max, run_idx, col_scratch):
    t = pl.program_id(0)

    @pl.when(t == 0)
    def _():
        col_scratch[...] = jax.lax.broadcasted_iota(jnp.int32, (R, BC_W), 1)

    @pl.when(t < NB_R)
    def _():
        x = x_ref[...]
        m = jnp.max(x, axis=1, keepdims=True)
        loc = jnp.argmax(x, axis=1).astype(jnp.int32).reshape(R, 1) + t * BC_R

        @pl.when(t == 0)
        def _():
            run_max[...] = m
            run_idx[...] = loc

        @pl.when(t > 0)
        def _():
            better = m > run_max[...]
            run_idx[...] = jnp.where(better, loc, run_idx[...])
            run_max[...] = jnp.maximum(m, run_max[...])

    @pl.when(t >= NB_R)
    def _():
        j = t - NB_R
        idx_s = run_idx[...] - j * BC_W
        out_ref[...] = (col_scratch[...] == idx_s).astype(jnp.float32)


@jax.jit
def kernel(Xsoft, P):
    del P
    return pl.pallas_call(
        _body,
        grid=(NB_R + NB_W,),
        in_specs=[
            pl.BlockSpec((R, BC_R), lambda t: (0, jnp.minimum(t, NB_R - 1))),
        ],
        out_specs=pl.BlockSpec(
            (R, BC_W), lambda t: (0, jnp.maximum(t - NB_R, 0))
        ),
        out_shape=jax.ShapeDtypeStruct((R, C), jnp.float32),
        scratch_shapes=[
            pltpu.VMEM((R, 1), jnp.float32),
            pltpu.VMEM((R, 1), jnp.int32),
            pltpu.VMEM((R, BC_W), jnp.int32),
        ],
    )(Xsoft)


# speedup vs baseline: 1.2263x; 1.0989x over previous
import jax
import jax.numpy as jnp
from jax.experimental import pallas as pl
from jax.experimental.pallas import tpu as pltpu

R = 128
C = 32768
BC_R = 8192
BC_W = 8192
NB_R = C // BC_R
NB_W = C // BC_W


def _body(x_ref, out_ref, run_max, run_idx, col_scratch):
    t = pl.program_id(0)

    @pl.when(t == 0)
    def _():
        col_scratch[...] = jax.lax.broadcasted_iota(jnp.int32, (R, BC_W), 1)

    @pl.when(t < NB_R)
    def _():
        x = x_ref[...]
        m = jnp.max(x, axis=1, keepdims=True)
        loc = jnp.argmax(x, axis=1).astype(jnp.int32).reshape(R, 1) + t * BC_R

        @pl.when(t == 0)
        def _():
            run_max[...] = m
            run_idx[...] = loc

        @pl.when(t > 0)
        def _():
            better = m > run_max[...]
            run_idx[...] = jnp.where(better, loc, run_idx[...])
            run_max[...] = jnp.maximum(m, run_max[...])

    @pl.when(t >= NB_R)
    def _():
        j = t - NB_R
        idx_s = run_idx[...] - j * BC_W
        out_ref[...] = (col_scratch[...] == idx_s).astype(jnp.float32)


@jax.jit
def kernel(Xsoft, P):
    del P
    return pl.pallas_call(
        _body,
        grid=(NB_R + NB_W,),
        in_specs=[
            pl.BlockSpec((R, BC_R), lambda t: (0, jnp.minimum(t, NB_R - 1))),
        ],
        out_specs=pl.BlockSpec(
            (R, BC_W), lambda t: (0, jnp.maximum(t - NB_R, 0))
        ),
        out_shape=jax.ShapeDtypeStruct((R, C), jnp.float32),
        scratch_shapes=[
            pltpu.VMEM((R, 1), jnp.float32),
            pltpu.VMEM((R, 1), jnp.int32),
            pltpu.VMEM((R, BC_W), jnp.int32),
        ],
    )(Xsoft)
